# baseline (speedup 1.0000x reference)
import functools

import jax
import jax.numpy as jnp
from jax import lax
from jax.experimental import pallas as pl
from jax.experimental.pallas import tpu as pltpu
from jax.experimental.pallas import tpu_sc as plsc

NUM_IMAGES = 100000
EMBED_DIM = 32
BATCH = 16384

_info = plsc.get_sparse_core_info()
_NC, _NS = _info.num_cores, _info.num_subcores
_NW = _NC * _NS
_LANE_TILES = (NUM_IMAGES + 127) // 128
_BASE_W = _LANE_TILES // _NW
_EXTRA = _LANE_TILES - _BASE_W * _NW
_MAX_W = _BASE_W + 1
_CHUNK = 2048
_NCHUNK = BATCH // _CHUNK
_NVEC = _CHUNK // 16
_NSLOT = 16
_JBITS = 14


@functools.partial(
    pl.kernel,
    mesh=plsc.VectorSubcoreMesh(core_axis_name="c", subcore_axis_name="s"),
    out_type=jax.ShapeDtypeStruct((BATCH * EMBED_DIM,), jnp.float32),
    scratch_types=[
        pltpu.VMEM((4, _MAX_W, 8, 128), jnp.float32),
        pltpu.VMEM((2, _CHUNK), jnp.int32),
        pltpu.VMEM((BATCH + 16,), jnp.int32),
        pltpu.VMEM((_NSLOT, EMBED_DIM), jnp.float32),
        pltpu.SemaphoreType.DMA,
        pltpu.SemaphoreType.DMA((2,)),
        pltpu.SemaphoreType.DMA((_NSLOT,)),
    ],
    compiler_params=pltpu.CompilerParams(
        use_tc_tiling_on_sc=True,
        needs_layout_passes=False,
        disable_bounds_checks=True,
    ),
)
def _lookup_kernel(table_t, idx_hbm, out_hbm, block_v, idx_v, hit_v, rows_v,
                   sem_blk, sem_idx, sem_out):
    wid = lax.axis_index("s") * _NC + lax.axis_index("c")
    c0 = wid * _BASE_W + jnp.minimum(wid, _EXTRA)
    wc = jnp.where(wid < _EXTRA, _BASE_W + 1, _BASE_W)
    lo = c0 * 128
    hi = (c0 + wc) * 128

    iota = lax.iota(jnp.int32, 16)
    d_lo = iota // 8
    s_lo = iota % 8
    zero16 = jnp.zeros((16,), jnp.int32)
    lo_v = jnp.full((16,), lo, jnp.int32)
    hi_v = jnp.full((16,), hi, jnp.int32)
    c0_v = jnp.full((16,), c0, jnp.int32)

    def blk_walk(ct, do_issue):
        for d in range(4):
            cp = pltpu.make_async_copy(
                table_t.at[pl.ds(d * 8, 8), pl.ds((c0 + ct) * 128, 128)],
                block_v.at[d, ct],
                sem_blk,
            )
            if do_issue:
                cp.start()
            else:
                cp.wait()
        return ct + 1

    def idx_chunk_copy(g):
        return pltpu.make_async_copy(
            idx_hbm.at[pl.ds(g * _CHUNK, _CHUNK)],
            idx_v.at[g % 2],
            sem_idx.at[g % 2],
        )

    idx_chunk_copy(0).start()
    lax.fori_loop(0, wc, lambda ct, _: blk_walk(ct, True), 0)

    def scan_chunk(g, cnt):
        idx_chunk_copy(g).wait()

        @pl.when(g + 1 < _NCHUNK)
        def _():
            idx_chunk_copy(g + 1).start()

        def scan_vec(v, cnt):
            c_vec = idx_v[g % 2, pl.ds(v * 16, 16)]
            m = (c_vec >= lo_v) & (c_vec < hi_v)
            n = plsc.all_reduce_population_count(m)[0]
            packed = jnp.left_shift(c_vec, _JBITS) | (
                iota + (g * _CHUNK + v * 16)
            )
            plsc.store_compressed(hit_v.at[pl.ds(cnt, 16)], packed, mask=m)
            return cnt + n

        return lax.fori_loop(0, _NVEC, scan_vec, cnt, unroll=4)

    total = lax.fori_loop(0, _NCHUNK, scan_chunk, jnp.int32(0))

    lax.fori_loop(0, wc, lambda ct, _: blk_walk(ct, False), 0)

    @pl.when(total > 0)
    def _():
        last = hit_v[pl.ds(total - 1, 16)]
        hit_v[pl.ds(total, 16)] = last.at[zero16].get(
            mode="promise_in_bounds"
        )

    total_pad = (total + 7) & ~7

    def hit_group(grp, _):
        for u in range(8):
            k = grp * 8 + u
            pk_vec = hit_v[pl.ds(k, 16)]
            pk_spl = pk_vec.at[zero16].get(mode="promise_in_bounds")
            qt_spl = jnp.right_shift(pk_spl, _JBITS + 7) - c0_v
            ql_spl = jnp.right_shift(pk_spl, _JBITS) & 127
            lo16 = plsc.load_gather(block_v, [d_lo, qt_spl, s_lo, ql_spl])
            hi16 = plsc.load_gather(block_v, [d_lo + 2, qt_spl, s_lo, ql_spl])
            slot = k & (_NSLOT - 1)
            slot_ref = rows_v.at[slot]

            @pl.when(k >= _NSLOT)
            def _():
                pltpu.make_async_copy(
                    out_hbm.at[pl.ds(0, EMBED_DIM)], slot_ref,
                    sem_out.at[slot],
                ).wait()

            slot_ref[pl.ds(0, 16)] = lo16
            slot_ref[pl.ds(16, 16)] = hi16
            j = pk_vec[0] & (BATCH - 1)
            pltpu.make_async_copy(
                slot_ref, out_hbm.at[pl.ds(j * EMBED_DIM, EMBED_DIM)],
                sem_out.at[slot],
            ).start()
        return 0

    lax.fori_loop(0, total_pad // 8, hit_group, 0)

    for s in range(_NSLOT):
        @pl.when(total_pad > s)
        def _():
            pltpu.make_async_copy(
                out_hbm.at[pl.ds(0, EMBED_DIM)], rows_v.at[s], sem_out.at[s]
            ).wait()


def kernel(camera_indices, embedding_weight):
    idx = camera_indices.astype(jnp.int32)
    flat = _lookup_kernel(embedding_weight.T, idx)
    return flat.reshape(BATCH, EMBED_DIM)


# speedup vs baseline: 1.0038x; 1.0038x over previous
import functools

import jax
import jax.numpy as jnp
from jax import lax
from jax.experimental import pallas as pl
from jax.experimental.pallas import tpu as pltpu
from jax.experimental.pallas import tpu_sc as plsc

NUM_IMAGES = 100000
EMBED_DIM = 32
BATCH = 16384

_info = plsc.get_sparse_core_info()
_NC, _NS = _info.num_cores, _info.num_subcores
_NW = _NC * _NS
_LANE_TILES = (NUM_IMAGES + 127) // 128
_BASE_W = _LANE_TILES // _NW
_EXTRA = _LANE_TILES - _BASE_W * _NW
_MAX_W = _BASE_W + 1
_CHUNK = 2048
_NCHUNK = BATCH // _CHUNK
_NVEC = _CHUNK // 16
_NSLOT = 16
_JBITS = 14


@functools.partial(
    pl.kernel,
    mesh=plsc.VectorSubcoreMesh(core_axis_name="c", subcore_axis_name="s"),
    out_type=jax.ShapeDtypeStruct((BATCH * EMBED_DIM,), jnp.float32),
    scratch_types=[
        pltpu.VMEM((4, _MAX_W, 8, 128), jnp.float32),
        pltpu.VMEM((2, _CHUNK), jnp.int32),
        pltpu.VMEM((BATCH + 16,), jnp.int32),
        pltpu.VMEM((_NSLOT, EMBED_DIM), jnp.float32),
        pltpu.SemaphoreType.DMA,
        pltpu.SemaphoreType.DMA((2,)),
        pltpu.SemaphoreType.DMA((_NSLOT,)),
    ],
    compiler_params=pltpu.CompilerParams(
        use_tc_tiling_on_sc=True,
        needs_layout_passes=False,
        disable_bounds_checks=True,
    ),
)
def _lookup_kernel(table_t, idx_hbm, out_hbm, block_v, idx_v, hit_v, rows_v,
                   sem_blk, sem_idx, sem_out):
    wid = lax.axis_index("s") * _NC + lax.axis_index("c")
    c0 = wid * _BASE_W + jnp.minimum(wid, _EXTRA)
    wc = jnp.where(wid < _EXTRA, _BASE_W + 1, _BASE_W)
    lo = c0 * 128
    hi = (c0 + wc) * 128

    iota = lax.iota(jnp.int32, 16)
    d_lo = iota // 8
    s_lo = iota % 8
    zero16 = jnp.zeros((16,), jnp.int32)
    lo_v = jnp.full((16,), lo, jnp.int32)
    hi_v = jnp.full((16,), hi, jnp.int32)
    c0_v = jnp.full((16,), c0, jnp.int32)

    def blk_walk(ct, do_issue):
        for d in range(4):
            cp = pltpu.make_async_copy(
                table_t.at[pl.ds(d * 8, 8), pl.ds((c0 + ct) * 128, 128)],
                block_v.at[d, ct],
                sem_blk,
            )
            if do_issue:
                cp.start()
            else:
                cp.wait()
        return ct + 1

    def idx_chunk_copy(g):
        return pltpu.make_async_copy(
            idx_hbm.at[pl.ds(g * _CHUNK, _CHUNK)],
            idx_v.at[g % 2],
            sem_idx.at[g % 2],
        )

    idx_chunk_copy(0).start()
    lax.fori_loop(0, wc, lambda ct, _: blk_walk(ct, True), 0)

    def scan_chunk(g, cnt):
        idx_chunk_copy(g).wait()

        @pl.when(g + 1 < _NCHUNK)
        def _():
            idx_chunk_copy(g + 1).start()

        def scan_vec(v, cnt):
            c_vec = idx_v[g % 2, pl.ds(v * 16, 16)]
            m = (c_vec >= lo_v) & (c_vec < hi_v)
            n = plsc.all_reduce_population_count(m)[0]
            packed = jnp.left_shift(c_vec, _JBITS) | (
                iota + (g * _CHUNK + v * 16)
            )
            plsc.store_compressed(hit_v.at[pl.ds(cnt, 16)], packed, mask=m)
            return cnt + n

        return lax.fori_loop(0, _NVEC, scan_vec, cnt, unroll=4)

    total = lax.fori_loop(0, _NCHUNK, scan_chunk, jnp.int32(0))

    lax.fori_loop(0, wc, lambda ct, _: blk_walk(ct, False), 0)

    @pl.when(total > 0)
    def _():
        last = hit_v[pl.ds(total - 1, 16)]
        hit_v[pl.ds(total, 16)] = last.at[zero16].get(
            mode="promise_in_bounds"
        )

    total_pad = (total + 3) & ~3

    def hit_group(grp, _):
        for u in range(4):
            k = grp * 4 + u
            pk_vec = hit_v[pl.ds(k, 16)]
            pk_spl = pk_vec.at[zero16].get(mode="promise_in_bounds")
            qt_spl = jnp.right_shift(pk_spl, _JBITS + 7) - c0_v
            ql_spl = jnp.right_shift(pk_spl, _JBITS) & 127
            lo16 = plsc.load_gather(block_v, [d_lo, qt_spl, s_lo, ql_spl])
            hi16 = plsc.load_gather(block_v, [d_lo + 2, qt_spl, s_lo, ql_spl])
            slot = k & (_NSLOT - 1)
            slot_ref = rows_v.at[slot]

            @pl.when(k >= _NSLOT)
            def _():
                pltpu.make_async_copy(
                    out_hbm.at[pl.ds(0, EMBED_DIM)], slot_ref,
                    sem_out.at[slot],
                ).wait()

            slot_ref[pl.ds(0, 16)] = lo16
            slot_ref[pl.ds(16, 16)] = hi16
            j = pk_vec[0] & (BATCH - 1)
            pltpu.make_async_copy(
                slot_ref, out_hbm.at[pl.ds(j * EMBED_DIM, EMBED_DIM)],
                sem_out.at[slot],
            ).start()
        return 0

    lax.fori_loop(0, total_pad // 4, hit_group, 0)

    for s in range(_NSLOT):
        @pl.when(total_pad > s)
        def _():
            pltpu.make_async_copy(
                out_hbm.at[pl.ds(0, EMBED_DIM)], rows_v.at[s], sem_out.at[s]
            ).wait()


def kernel(camera_indices, embedding_weight):
    idx = camera_indices.astype(jnp.int32)
    flat = _lookup_kernel(embedding_weight.T, idx)
    return flat.reshape(BATCH, EMBED_DIM)


# speedup vs baseline: 1.0122x; 1.0084x over previous
import functools

import jax
import jax.numpy as jnp
from jax import lax
from jax.experimental import pallas as pl
from jax.experimental.pallas import tpu as pltpu
from jax.experimental.pallas import tpu_sc as plsc

NUM_IMAGES = 100000
EMBED_DIM = 32
BATCH = 16384

_info = plsc.get_sparse_core_info()
_NC, _NS = _info.num_cores, _info.num_subcores
_NW = _NC * _NS
_LANE_TILES = (NUM_IMAGES + 127) // 128
_BASE_W = _LANE_TILES // _NW
_EXTRA = _LANE_TILES - _BASE_W * _NW
_MAX_W = _BASE_W + 1
_CHUNK = 4096
_NCHUNK = BATCH // _CHUNK
_NVEC = _CHUNK // 16
_NSLOT = 16
_JBITS = 14


@functools.partial(
    pl.kernel,
    mesh=plsc.VectorSubcoreMesh(core_axis_name="c", subcore_axis_name="s"),
    out_type=jax.ShapeDtypeStruct((BATCH * EMBED_DIM,), jnp.float32),
    scratch_types=[
        pltpu.VMEM((4, _MAX_W, 8, 128), jnp.float32),
        pltpu.VMEM((2, _CHUNK), jnp.int32),
        pltpu.VMEM((BATCH + 16,), jnp.int32),
        pltpu.VMEM((_NSLOT, EMBED_DIM), jnp.float32),
        pltpu.SemaphoreType.DMA,
        pltpu.SemaphoreType.DMA((2,)),
        pltpu.SemaphoreType.DMA((_NSLOT,)),
    ],
    compiler_params=pltpu.CompilerParams(
        use_tc_tiling_on_sc=True,
        needs_layout_passes=False,
        disable_bounds_checks=True,
    ),
)
def _lookup_kernel(table_t, idx_hbm, out_hbm, block_v, idx_v, hit_v, rows_v,
                   sem_blk, sem_idx, sem_out):
    wid = lax.axis_index("s") * _NC + lax.axis_index("c")
    c0 = wid * _BASE_W + jnp.minimum(wid, _EXTRA)
    wc = jnp.where(wid < _EXTRA, _BASE_W + 1, _BASE_W)
    lo = c0 * 128
    hi = (c0 + wc) * 128

    iota = lax.iota(jnp.int32, 16)
    d_lo = iota // 8
    s_lo = iota % 8
    zero16 = jnp.zeros((16,), jnp.int32)
    lo_v = jnp.full((16,), lo, jnp.int32)
    hi_v = jnp.full((16,), hi, jnp.int32)
    c0_v = jnp.full((16,), c0, jnp.int32)

    def blk_walk(ct, do_issue):
        for d in range(4):
            cp = pltpu.make_async_copy(
                table_t.at[pl.ds(d * 8, 8), pl.ds((c0 + ct) * 128, 128)],
                block_v.at[d, ct],
                sem_blk,
            )
            if do_issue:
                cp.start()
            else:
                cp.wait()
        return ct + 1

    def idx_chunk_copy(g):
        return pltpu.make_async_copy(
            idx_hbm.at[pl.ds(g * _CHUNK, _CHUNK)],
            idx_v.at[g % 2],
            sem_idx.at[g % 2],
        )

    idx_chunk_copy(0).start()
    lax.fori_loop(0, wc, lambda ct, _: blk_walk(ct, True), 0)

    def scan_chunk(g, cnt):
        idx_chunk_copy(g).wait()

        @pl.when(g + 1 < _NCHUNK)
        def _():
            idx_chunk_copy(g + 1).start()

        def scan_vec(v, cnt):
            c_vec = idx_v[g % 2, pl.ds(v * 16, 16)]
            m = (c_vec >= lo_v) & (c_vec < hi_v)
            n = plsc.all_reduce_population_count(m)[0]
            packed = jnp.left_shift(c_vec, _JBITS) | (
                iota + (g * _CHUNK + v * 16)
            )
            plsc.store_compressed(hit_v.at[pl.ds(cnt, 16)], packed, mask=m)
            return cnt + n

        return lax.fori_loop(0, _NVEC, scan_vec, cnt, unroll=4)

    total = lax.fori_loop(0, _NCHUNK, scan_chunk, jnp.int32(0))

    lax.fori_loop(0, wc, lambda ct, _: blk_walk(ct, False), 0)

    @pl.when(total > 0)
    def _():
        last = hit_v[pl.ds(total - 1, 16)]
        hit_v[pl.ds(total, 16)] = last.at[zero16].get(
            mode="promise_in_bounds"
        )

    total_pad = (total + 3) & ~3

    def hit_group(grp, _):
        for u in range(4):
            k = grp * 4 + u
            pk_vec = hit_v[pl.ds(k, 16)]
            pk_spl = pk_vec.at[zero16].get(mode="promise_in_bounds")
            qt_spl = jnp.right_shift(pk_spl, _JBITS + 7) - c0_v
            ql_spl = jnp.right_shift(pk_spl, _JBITS) & 127
            lo16 = plsc.load_gather(block_v, [d_lo, qt_spl, s_lo, ql_spl])
            hi16 = plsc.load_gather(block_v, [d_lo + 2, qt_spl, s_lo, ql_spl])
            slot = k & (_NSLOT - 1)
            slot_ref = rows_v.at[slot]

            @pl.when(k >= _NSLOT)
            def _():
                pltpu.make_async_copy(
                    out_hbm.at[pl.ds(0, EMBED_DIM)], slot_ref,
                    sem_out.at[slot],
                ).wait()

            slot_ref[pl.ds(0, 16)] = lo16
            slot_ref[pl.ds(16, 16)] = hi16
            j = pk_vec[0] & (BATCH - 1)
            pltpu.make_async_copy(
                slot_ref, out_hbm.at[pl.ds(j * EMBED_DIM, EMBED_DIM)],
                sem_out.at[slot],
            ).start()
        return 0

    lax.fori_loop(0, total_pad // 4, hit_group, 0)

    for s in range(_NSLOT):
        @pl.when(total_pad > s)
        def _():
            pltpu.make_async_copy(
                out_hbm.at[pl.ds(0, EMBED_DIM)], rows_v.at[s], sem_out.at[s]
            ).wait()


def kernel(camera_indices, embedding_weight):
    idx = camera_indices.astype(jnp.int32)
    flat = _lookup_kernel(embedding_weight.T, idx)
    return flat.reshape(BATCH, EMBED_DIM)
